# baseline (speedup 1.0000x reference)
import functools

import jax
import jax.numpy as jnp
from jax import lax
from jax.experimental import pallas as pl
from jax.experimental.pallas import tpu as pltpu
from jax.experimental.pallas import tpu_sc as plsc

_L16 = 16
_NC = 2
_NS = 16
_NW = _NC * _NS
_NL = 32
_ND = 32
_NE = 32
_B = 1024
_BW = _B // _NW
_DG = 4
_CHUNK = _DG * _BW
_CPL = _ND // _DG
_NCHUNK = _NL * _CPL
_BC = _BW // _L16


def _rsqrt_nr(x):
    i = plsc.bitcast(x, jnp.int32)
    i = jnp.int32(0x5F3759DF) - (i >> 1)
    y = plsc.bitcast(i, jnp.float32)
    for _ in range(3):
        y = y * (jnp.float32(1.5) - jnp.float32(0.5) * x * y * y)
    return y


def _make_sc_kernel():
    mesh = plsc.VectorSubcoreMesh(core_axis_name="c", subcore_axis_name="s")
    f32 = jnp.float32

    @functools.partial(
        pl.kernel,
        mesh=mesh,
        compiler_params=pltpu.CompilerParams(needs_layout_passes=False,
                                             use_tc_tiling_on_sc=False),
        out_type=jax.ShapeDtypeStruct((_NL, _ND, 4, 8, 8, 128), f32),
        scratch_types=[
            pltpu.VMEM((_NL, 4, 8, _BW), f32),
            pltpu.VMEM((1, _CHUNK), jnp.int32),
            pltpu.VMEM((1, _CHUNK), jnp.int32),
            pltpu.VMEM((_CHUNK, _NE), f32),
            pltpu.VMEM((_CHUNK, _NE), f32),
            pltpu.VMEM((2, _ND, 4, 8, _BW), f32),
            pltpu.SemaphoreType.DMA,
            pltpu.SemaphoreType.DMA,
            pltpu.SemaphoreType.DMA,
        ],
    )
    def sc_kernel(idx_hbm, tab_hbm, out_hbm, idx_n, ib0, ib1, ea0, ea1,
                  tbuf, sg0, sg1, so):
        w = lax.axis_index("s") * _NC + lax.axis_index("c")
        pltpu.sync_copy(
            idx_hbm.at[:, :, w >> 2, :, pl.ds((w & 3) * _BW, _BW)], idx_n)

        iota16 = lax.iota(jnp.int32, _L16)
        zero = jnp.zeros((_L16,), f32)
        inv_n = jnp.float32(1.0 / (_ND * _NE))
        bufs = ((ib0, ea0, sg0), (ib1, ea1, sg1))

        def stage_and_issue(c, par):
            ib, ea, sg = bufs[par]
            l = c >> 3
            dgrp = c & 7
            for v in range(8):
                d = dgrp * _DG + (v >> 1)
                il = jnp.full((_L16,), l, jnp.int32)
                ir = jnp.full((_L16,), d >> 3, jnp.int32)
                is_ = jnp.full((_L16,), d & 7, jnp.int32)
                ik = iota16 + (v & 1) * _L16
                vals = plsc.load_gather(idx_n, [il, ir, is_, ik])
                ib[0, pl.ds(v * _L16, _L16)] = plsc.bitcast(vals, jnp.int32)
            pltpu.async_copy(tab_hbm.at[ib.at[0]], ea, sg)

        for par in (0, 1):
            stage_and_issue(par, par)

        out_drain = out_hbm.at[0, :, :, 0, :, pl.ds(0, _BW)]

        def chunk(c, par, accs):
            _, ea, sg = bufs[par]
            l = c >> 3
            lpar = l & 1
            first = (c & 7) == 0
            s0, s1, q0, q1 = accs
            s0 = jnp.where(first, zero, s0)
            s1 = jnp.where(first, zero, s1)
            q0 = jnp.where(first, zero, q0)
            q1 = jnp.where(first, zero, q1)

            @pl.when(jnp.logical_and(first, c >= 2 * _CPL))
            def _():
                pltpu.make_async_copy(tbuf.at[0], out_drain, so).wait()

            pltpu.make_async_copy(tab_hbm.at[pl.ds(0, _CHUNK)], ea, sg).wait()

            dgrp = c & 7
            accs = [s0, s1, q0, q1]
            for dl in range(_DG):
                d = dgrp * _DG + dl
                for e in range(_NE):
                    col = jnp.full((_L16,), e, jnp.int32)
                    for bc in range(_BC):
                        rows = iota16 + (dl * _BW + bc * _L16)
                        v = plsc.load_gather(ea, [rows, col])
                        tbuf[lpar, d, e >> 3, e & 7,
                             pl.ds(bc * _L16, _L16)] = v
                        accs[bc] = accs[bc] + v
                        accs[_BC + bc] = accs[_BC + bc] + v * v

            @pl.when(c < _NCHUNK - 2)
            def _():
                stage_and_issue(c + 2, par)

            @pl.when((c & 7) == 7)
            def _():
                scs, shs = [], []
                for bc in range(_BC):
                    mean = accs[bc] * inv_n
                    var = jnp.maximum(
                        accs[_BC + bc] * inv_n - mean * mean, jnp.float32(0.0))
                    sc = _rsqrt_nr(var + jnp.float32(1e-5))
                    scs.append(sc)
                    shs.append(mean * sc)

                def dnorm(d, carry):
                    for e in range(_NE):
                        for bc in range(_BC):
                            sl = (lpar, d, e >> 3, e & 7,
                                  pl.ds(bc * _L16, _L16))
                            tbuf[sl] = tbuf[sl] * scs[bc] - shs[bc]
                    return carry

                lax.fori_loop(0, _ND, dnorm, 0)
                pltpu.async_copy(
                    tbuf.at[lpar],
                    out_hbm.at[l, :, :, w >> 2, :, pl.ds((w & 3) * _BW, _BW)],
                    so)

            return tuple(accs)

        def pair(j, accs):
            for par in (0, 1):
                accs = chunk(j * 2 + par, par, accs)
            return accs

        lax.fori_loop(0, _NCHUNK // 2, pair, (zero, zero, zero, zero))

        for _p in range(2):
            pltpu.make_async_copy(tbuf.at[0], out_drain, so).wait()

    return sc_kernel


_sc_kernel = _make_sc_kernel()


def kernel(x, emb_weight, pos_weight):
    tab = emb_weight + pos_weight
    xt = (jnp.transpose(x, (1, 2, 0))
          .reshape(_NL, 4, 8, 8, 128)
          .transpose(0, 1, 3, 2, 4))
    xt = lax.bitcast_convert_type(xt, jnp.float32)
    o6 = _sc_kernel(xt, tab)
    return jnp.transpose(o6, (3, 5, 0, 1, 2, 4)).reshape(_B, _NL, _ND, _NE)


# speedup vs baseline: 1.4323x; 1.4323x over previous
import functools

import jax
import jax.numpy as jnp
from jax import lax
from jax.experimental import pallas as pl
from jax.experimental.pallas import tpu as pltpu
from jax.experimental.pallas import tpu_sc as plsc

_L16 = 16
_NC = 2
_NS = 16
_NW = _NC * _NS
_NL = 32
_ND = 32
_NE = 32
_B = 1024
_BW = _B // _NW
_DG = 4
_CHUNK = _DG * _BW
_CPL = _ND // _DG
_NCHUNK = _NL * _CPL
_BC = _BW // _L16


def _rsqrt_nr(x):
    i = plsc.bitcast(x, jnp.int32)
    i = jnp.int32(0x5F3759DF) - (i >> 1)
    y = plsc.bitcast(i, jnp.float32)
    for _ in range(3):
        y = y * (jnp.float32(1.5) - jnp.float32(0.5) * x * y * y)
    return y


def _make_sc_kernel():
    mesh = plsc.VectorSubcoreMesh(core_axis_name="c", subcore_axis_name="s")
    f32 = jnp.float32

    @functools.partial(
        pl.kernel,
        mesh=mesh,
        compiler_params=pltpu.CompilerParams(needs_layout_passes=False,
                                             use_tc_tiling_on_sc=False),
        out_type=jax.ShapeDtypeStruct((_NL, _ND, 4, 8, 8, 128), f32),
        scratch_types=[
            pltpu.VMEM((_NL, 4, 8, _BW), f32),
            pltpu.VMEM((1, _CHUNK), jnp.int32),
            pltpu.VMEM((1, _CHUNK), jnp.int32),
            pltpu.VMEM((_CHUNK, _NE), f32),
            pltpu.VMEM((_CHUNK, _NE), f32),
            pltpu.VMEM((2, _ND, 4, 8, _BW), f32),
            pltpu.SemaphoreType.DMA,
            pltpu.SemaphoreType.DMA,
            pltpu.SemaphoreType.DMA,
        ],
    )
    def sc_kernel(idx_hbm, tab_hbm, out_hbm, idx_n, ib0, ib1, ea0, ea1,
                  tbuf, sg0, sg1, so):
        w = lax.axis_index("s") * _NC + lax.axis_index("c")
        pltpu.sync_copy(
            idx_hbm.at[:, :, w >> 2, :, pl.ds((w & 3) * _BW, _BW)], idx_n)

        iota16 = lax.iota(jnp.int32, _L16)
        zero = jnp.zeros((_L16,), f32)
        inv_n = jnp.float32(1.0 / (_ND * _NE))
        bufs = ((ib0, ea0, sg0), (ib1, ea1, sg1))

        def stage_and_issue(c, par):
            ib, ea, sg = bufs[par]
            l = c >> 3
            dgrp = c & 7
            vals = []
            for v in range(8):
                d = dgrp * _DG + (v >> 1)
                il = jnp.full((_L16,), l, jnp.int32)
                ir = jnp.full((_L16,), d >> 3, jnp.int32)
                is_ = jnp.full((_L16,), d & 7, jnp.int32)
                ik = iota16 + (v & 1) * _L16
                vals.append(plsc.load_gather(idx_n, [il, ir, is_, ik]))
            for v in range(8):
                ib[0, pl.ds(v * _L16, _L16)] = plsc.bitcast(vals[v], jnp.int32)
            pltpu.async_copy(tab_hbm.at[ib.at[0]], ea, sg)

        for par in (0, 1):
            stage_and_issue(par, par)

        out_drain = out_hbm.at[0, :, :, 0, :, pl.ds(0, _BW)]

        def chunk(c, par, accs):
            _, ea, sg = bufs[par]
            l = c >> 3
            lpar = l & 1
            first = (c & 7) == 0
            s0, s1, q0, q1 = accs
            s0 = jnp.where(first, zero, s0)
            s1 = jnp.where(first, zero, s1)
            q0 = jnp.where(first, zero, q0)
            q1 = jnp.where(first, zero, q1)

            @pl.when(jnp.logical_and(first, c >= 2 * _CPL))
            def _():
                pltpu.make_async_copy(tbuf.at[0], out_drain, so).wait()

            pltpu.make_async_copy(tab_hbm.at[pl.ds(0, _CHUNK)], ea, sg).wait()

            dgrp = c & 7
            accs = [s0, s1, q0, q1]
            for dl in range(_DG):
                d = dgrp * _DG + dl
                for eg in range(_NE // 8):
                    vecs = []
                    for e8 in range(8):
                        col = jnp.full((_L16,), eg * 8 + e8, jnp.int32)
                        for bc in range(_BC):
                            rows = iota16 + (dl * _BW + bc * _L16)
                            vecs.append(plsc.load_gather(ea, [rows, col]))
                    i = 0
                    for e8 in range(8):
                        e = eg * 8 + e8
                        for bc in range(_BC):
                            v = vecs[i]
                            i += 1
                            tbuf[lpar, d, e >> 3, e & 7,
                                 pl.ds(bc * _L16, _L16)] = v
                            accs[bc] = accs[bc] + v
                            accs[_BC + bc] = accs[_BC + bc] + v * v

            @pl.when(c < _NCHUNK - 2)
            def _():
                stage_and_issue(c + 2, par)

            @pl.when((c & 7) == 7)
            def _():
                scs, shs = [], []
                for bc in range(_BC):
                    mean = accs[bc] * inv_n
                    var = jnp.maximum(
                        accs[_BC + bc] * inv_n - mean * mean, jnp.float32(0.0))
                    sc = _rsqrt_nr(var + jnp.float32(1e-5))
                    scs.append(sc)
                    shs.append(mean * sc)

                def dnorm(d, carry):
                    for eg in range(_NE // 8):
                        vecs = []
                        for e8 in range(8):
                            e = eg * 8 + e8
                            for bc in range(_BC):
                                sl = (lpar, d, e >> 3, e & 7,
                                      pl.ds(bc * _L16, _L16))
                                vecs.append(tbuf[sl] * scs[bc] - shs[bc])
                        i = 0
                        for e8 in range(8):
                            e = eg * 8 + e8
                            for bc in range(_BC):
                                tbuf[lpar, d, e >> 3, e & 7,
                                     pl.ds(bc * _L16, _L16)] = vecs[i]
                                i += 1
                    return carry

                lax.fori_loop(0, _ND, dnorm, 0)
                pltpu.async_copy(
                    tbuf.at[lpar],
                    out_hbm.at[l, :, :, w >> 2, :, pl.ds((w & 3) * _BW, _BW)],
                    so)

            return tuple(accs)

        def pair(j, accs):
            for par in (0, 1):
                accs = chunk(j * 2 + par, par, accs)
            return accs

        lax.fori_loop(0, _NCHUNK // 2, pair, (zero, zero, zero, zero))

        for _p in range(2):
            pltpu.make_async_copy(tbuf.at[0], out_drain, so).wait()

    return sc_kernel


_sc_kernel = _make_sc_kernel()


def kernel(x, emb_weight, pos_weight):
    tab = emb_weight + pos_weight
    xt = (jnp.transpose(x, (1, 2, 0))
          .reshape(_NL, 4, 8, 8, 128)
          .transpose(0, 1, 3, 2, 4))
    xt = lax.bitcast_convert_type(xt, jnp.float32)
    o6 = _sc_kernel(xt, tab)
    return jnp.transpose(o6, (3, 5, 0, 1, 2, 4)).reshape(_B, _NL, _ND, _NE)
